# Initial kernel scaffold; baseline (speedup 1.0000x reference)
import jax
import jax.numpy as jnp
from jax.experimental import pallas as pl


def kernel(x, edge_index, W, b):
    raise NotImplementedError("write your pallas kernel here")


# speedup vs baseline: 16.3335x; 16.3335x over previous
import functools

import jax
import jax.numpy as jnp
from jax import lax
from jax.experimental import pallas as pl
from jax.experimental.pallas import tpu as pltpu
from jax.experimental.pallas import tpu_sc as plsc

N_NODES = 10000
N_EDGES = 320000
D = 128

NC = 2
NS = 16
NW = NC * NS
E_PER_TILE = N_EDGES // NW
CHUNK = 80
NCHUNK = E_PER_TILE // CHUNK
N_PAD = 10240
STRIPE = N_PAD // NS
NCHUNK_F = N_EDGES // NS // CHUNK
IBLK = 25
NIB = NCHUNK_F // IBLK
OWN = 5000
OWN_PAD = 5120
ACC_ROWS = 5128
AGG_STRIPE = OWN_PAD // NS
ROW_CHUNK = 80


def _deg_body(dst_hbm, deg_hbm, idx_v, ones_v, stripe_v, acc_sh):
    c = lax.axis_index("c")
    s = lax.axis_index("s")
    for i in range(CHUNK // 16):
        ones_v[pl.ds(i * 16, 16)] = jnp.ones((16,), jnp.float32)
    for i in range(640 // 16):
        stripe_v[pl.ds(i * 16, 16)] = jnp.zeros((16,), jnp.float32)
    pltpu.sync_copy(stripe_v, acc_sh.at[pl.ds(s * 640, 640)])
    plsc.subcore_barrier()
    pltpu.sync_copy(dst_hbm.at[c, s], idx_v)

    def chunk(j, carry):
        pltpu.sync_copy(ones_v, acc_sh.at[idx_v.at[j]], add=True)
        return carry

    lax.fori_loop(0, NCHUNK, chunk, 0)
    plsc.subcore_barrier()
    pltpu.sync_copy(acc_sh.at[pl.ds(s * 640, 640)], stripe_v)
    pltpu.sync_copy(stripe_v, deg_hbm.at[c, pl.ds(s * 640, 640)])


@jax.jit
def _deg_kernel(dst4):
    mesh = plsc.VectorSubcoreMesh(core_axis_name="c", subcore_axis_name="s")
    return pl.kernel(
        _deg_body,
        out_type=jax.ShapeDtypeStruct((NC, N_PAD), jnp.float32),
        mesh=mesh,
        scratch_types=[
            pltpu.VMEM((NCHUNK, CHUNK), jnp.int32),
            pltpu.VMEM((CHUNK,), jnp.float32),
            pltpu.VMEM((640,), jnp.float32),
            pltpu.VMEM_SHARED((N_PAD,), jnp.float32),
        ],
    )(dst4)


def _agg_body(y_hbm, src_hbm, dst_hbm, agg_hbm,
              sidx_v, didx_v, rows_v, zbuf, acc_sh, sem):
    c = lax.axis_index("c")
    s = lax.axis_index("s")
    base = s * AGG_STRIPE
    lo = c * OWN

    def zrow(r, carry):
        for k in range(D // 16):
            zbuf[r, pl.ds(k * 16, 16)] = jnp.zeros((16,), jnp.float32)
        return carry

    lax.fori_loop(0, ROW_CHUNK, zrow, 0)
    for i in range(AGG_STRIPE // ROW_CHUNK):
        pltpu.sync_copy(zbuf, acc_sh.at[pl.ds(base + i * ROW_CHUNK, ROW_CHUNK)])
    plsc.subcore_barrier()

    def outer(ib, carry):
        pltpu.sync_copy(src_hbm.at[s, ib], sidx_v)
        pltpu.sync_copy(dst_hbm.at[s, ib], didx_v)

        def remap(t, carry2):
            j = t // (CHUNK // 16)
            k = t % (CHUNK // 16)
            d = didx_v[j, pl.ds(k * 16, 16)] - lo
            ok = jnp.logical_and(d >= 0, d < OWN)
            didx_v[j, pl.ds(k * 16, 16)] = jnp.where(ok, d, OWN_PAD)
            return carry2

        lax.fori_loop(0, IBLK * (CHUNK // 16), remap, 0)

        def chunk(j, carry2):
            pltpu.async_copy(y_hbm.at[sidx_v.at[j]], rows_v, sem).wait()
            pltpu.sync_copy(rows_v, acc_sh.at[didx_v.at[j]], add=True)
            return carry2

        lax.fori_loop(0, IBLK, chunk, 0)
        return carry

    lax.fori_loop(0, NIB, outer, 0)
    plsc.subcore_barrier()
    for i in range(AGG_STRIPE // ROW_CHUNK):
        off = base + i * ROW_CHUNK
        pltpu.sync_copy(acc_sh.at[pl.ds(off, ROW_CHUNK)], zbuf)
        pltpu.sync_copy(zbuf, agg_hbm.at[c, pl.ds(off, ROW_CHUNK)])


@jax.jit
def _agg_kernel(y, src3, dst3):
    mesh = plsc.VectorSubcoreMesh(core_axis_name="c", subcore_axis_name="s")
    return pl.kernel(
        _agg_body,
        out_type=jax.ShapeDtypeStruct((NC, OWN_PAD, D), jnp.float32),
        mesh=mesh,
        scratch_types=[
            pltpu.VMEM((IBLK, CHUNK), jnp.int32),
            pltpu.VMEM((IBLK, CHUNK), jnp.int32),
            pltpu.VMEM((CHUNK, D), jnp.float32),
            pltpu.VMEM((ROW_CHUNK, D), jnp.float32),
            pltpu.VMEM_SHARED((ACC_ROWS, D), jnp.float32),
            pltpu.SemaphoreType.DMA,
        ],
    )(y, src3, dst3)


def _mm_body(x_ref, w_ref, degp_ref, y_ref):
    xw = jnp.dot(x_ref[...], w_ref[...], preferred_element_type=jnp.float32)
    deg = degp_ref[0, 0, :] + degp_ref[0, 1, :] + 1.0
    dis = lax.rsqrt(deg)
    y_ref[...] = xw * dis[:, None]


@jax.jit
def _mm_kernel(x, W, degp5):
    rb = 2000
    grid = N_NODES // rb
    return pl.pallas_call(
        _mm_body,
        grid=(grid,),
        in_specs=[
            pl.BlockSpec((rb, D), lambda j: (j, 0)),
            pl.BlockSpec((D, D), lambda j: (0, 0)),
            pl.BlockSpec((1, NC, rb), lambda j: (j, 0, 0)),
        ],
        out_specs=pl.BlockSpec((rb, D), lambda j: (j, 0)),
        out_shape=jax.ShapeDtypeStruct((N_NODES, D), jnp.float32),
    )(x, W, degp5)


def _fin_body(agg_ref, y_ref, degp_ref, b_ref, out_ref):
    deg = degp_ref[0, 0, :] + degp_ref[0, 1, :] + 1.0
    dis = lax.rsqrt(deg)
    out_ref[...] = jnp.maximum(
        dis[:, None] * (agg_ref[0] + y_ref[...]) + b_ref[...], 0.0)


@jax.jit
def _fin_kernel(agg, y, degp10, b2):
    rb = 1000
    half = OWN // rb
    return pl.pallas_call(
        _fin_body,
        grid=(N_NODES // rb,),
        in_specs=[
            pl.BlockSpec((1, rb, D), lambda j: (j // half, j % half, 0)),
            pl.BlockSpec((rb, D), lambda j: (j, 0)),
            pl.BlockSpec((1, NC, rb), lambda j: (j, 0, 0)),
            pl.BlockSpec((1, D), lambda j: (0, 0)),
        ],
        out_specs=pl.BlockSpec((rb, D), lambda j: (j, 0)),
        out_shape=jax.ShapeDtypeStruct((N_NODES, D), jnp.float32),
    )(agg, y, degp10, b2)


def kernel(x, edge_index, W, b):
    ei = edge_index.astype(jnp.int32)
    src4 = ei[0].reshape(NC, NS, NCHUNK, CHUNK)
    dst4 = ei[1].reshape(NC, NS, NCHUNK, CHUNK)
    src3 = ei[0].reshape(NS, NIB, IBLK, CHUNK)
    dst3 = ei[1].reshape(NS, NIB, IBLK, CHUNK)
    deg_part = _deg_kernel(dst4)
    degp = deg_part[:, :N_NODES]
    degp5 = jnp.swapaxes(degp.reshape(NC, 5, 2000), 0, 1)
    degp10 = jnp.swapaxes(degp.reshape(NC, 10, 1000), 0, 1)
    y = _mm_kernel(x, W, degp5)
    agg = _agg_kernel(y, src3, dst3)
    return _fin_kernel(agg, y, degp10, b.reshape(1, D))


# speedup vs baseline: 24.3881x; 1.4931x over previous
import functools

import jax
import jax.numpy as jnp
from jax import lax
from jax.experimental import pallas as pl
from jax.experimental.pallas import tpu as pltpu
from jax.experimental.pallas import tpu_sc as plsc

N_NODES = 10000
N_EDGES = 320000
D = 128

NC = 2
NS = 16
NW = NC * NS
E_PER_TILE = N_EDGES // NW
CHUNK = 80
NCHUNK = E_PER_TILE // CHUNK
N_PAD = 10240
STRIPE = N_PAD // NS
NCHUNK_F = N_EDGES // NS // CHUNK
IBLK = 25
NIB = NCHUNK_F // IBLK
NBUF = 5
PREF = 2
LAG = 2
OWN = 5000
OWN_PAD = 5120
ACC_ROWS = 5128
AGG_STRIPE = OWN_PAD // NS
ROW_CHUNK = 80


def _deg_body(dst_hbm, deg_hbm, idx_v, ones_v, stripe_v, acc_sh):
    c = lax.axis_index("c")
    s = lax.axis_index("s")
    for i in range(CHUNK // 16):
        ones_v[pl.ds(i * 16, 16)] = jnp.ones((16,), jnp.float32)
    for i in range(640 // 16):
        stripe_v[pl.ds(i * 16, 16)] = jnp.zeros((16,), jnp.float32)
    pltpu.sync_copy(stripe_v, acc_sh.at[pl.ds(s * 640, 640)])
    plsc.subcore_barrier()
    pltpu.sync_copy(dst_hbm.at[c, s], idx_v)

    def chunk(j, carry):
        pltpu.sync_copy(ones_v, acc_sh.at[idx_v.at[j]], add=True)
        return carry

    lax.fori_loop(0, NCHUNK, chunk, 0)
    plsc.subcore_barrier()
    pltpu.sync_copy(acc_sh.at[pl.ds(s * 640, 640)], stripe_v)
    pltpu.sync_copy(stripe_v, deg_hbm.at[c, pl.ds(s * 640, 640)])


@jax.jit
def _deg_kernel(dst4):
    mesh = plsc.VectorSubcoreMesh(core_axis_name="c", subcore_axis_name="s")
    return pl.kernel(
        _deg_body,
        out_type=jax.ShapeDtypeStruct((NC, N_PAD), jnp.float32),
        mesh=mesh,
        scratch_types=[
            pltpu.VMEM((NCHUNK, CHUNK), jnp.int32),
            pltpu.VMEM((CHUNK,), jnp.float32),
            pltpu.VMEM((640,), jnp.float32),
            pltpu.VMEM_SHARED((N_PAD,), jnp.float32),
        ],
    )(dst4)


def _agg_body(y_hbm, src_hbm, dst_hbm, agg_hbm,
              sidx_v, didx_v, rows_v, acc_sh, *sems):
    isem = sems[0]
    gsem = sems[1:1 + NBUF]
    ssem = sems[1 + NBUF:1 + 2 * NBUF]
    c = lax.axis_index("c")
    s = lax.axis_index("s")
    base = s * AGG_STRIPE
    lo = c * OWN

    def zrow(r, carry):
        for k in range(D // 16):
            rows_v[0, r, pl.ds(k * 16, 16)] = jnp.zeros((16,), jnp.float32)
        return carry

    lax.fori_loop(0, ROW_CHUNK, zrow, 0)
    for i in range(AGG_STRIPE // ROW_CHUNK):
        pltpu.sync_copy(rows_v.at[0],
                        acc_sh.at[pl.ds(base + i * ROW_CHUNK, ROW_CHUNK)])
    plsc.subcore_barrier()

    pltpu.async_copy(src_hbm.at[s, 0], sidx_v.at[0], isem)
    pltpu.async_copy(dst_hbm.at[s, 0], didx_v.at[0], isem)

    def block(ib, carry):
        ib2 = lax.rem(ib, 2)
        pltpu.make_async_copy(src_hbm.at[s, ib], sidx_v.at[ib2], isem).wait()
        pltpu.make_async_copy(dst_hbm.at[s, ib], didx_v.at[ib2], isem).wait()

        gh = {}
        for j in range(PREF):
            gh[j] = pltpu.async_copy(
                y_hbm.at[sidx_v.at[ib2, j]], rows_v.at[j % NBUF],
                gsem[j % NBUF])

        @pl.when(ib < NIB - 1)
        def _():
            nxt = lax.rem(ib + 1, 2)
            pltpu.async_copy(src_hbm.at[s, ib + 1], sidx_v.at[nxt], isem)
            pltpu.async_copy(dst_hbm.at[s, ib + 1], didx_v.at[nxt], isem)

        def remap(t, carry2):
            j = t // (CHUNK // 16)
            k = t % (CHUNK // 16)
            d = didx_v[ib2, j, pl.ds(k * 16, 16)] - lo
            ok = jnp.logical_and(d >= 0, d < OWN)
            didx_v[ib2, j, pl.ds(k * 16, 16)] = jnp.where(ok, d, OWN_PAD)
            return carry2

        lax.fori_loop(0, IBLK * (CHUNK // 16), remap, 0)

        sh = {}
        for j in range(IBLK):
            b = j % NBUF
            gh[j].wait()
            sh[j] = pltpu.async_copy(
                rows_v.at[b], acc_sh.at[didx_v.at[ib2, j]], ssem[b], add=True)
            if j - LAG >= 0:
                sh[j - LAG].wait()
            nj = j + PREF
            if nj < IBLK:
                nb = nj % NBUF
                gh[nj] = pltpu.async_copy(
                    y_hbm.at[sidx_v.at[ib2, nj]], rows_v.at[nb], gsem[nb])
        for j in range(IBLK - LAG, IBLK):
            sh[j].wait()
        return carry

    lax.fori_loop(0, NIB, block, 0)
    plsc.subcore_barrier()
    for i in range(AGG_STRIPE // ROW_CHUNK):
        off = base + i * ROW_CHUNK
        pltpu.sync_copy(acc_sh.at[pl.ds(off, ROW_CHUNK)], rows_v.at[0])
        pltpu.sync_copy(rows_v.at[0], agg_hbm.at[c, pl.ds(off, ROW_CHUNK)])


@jax.jit
def _agg_kernel(y, src3, dst3):
    mesh = plsc.VectorSubcoreMesh(core_axis_name="c", subcore_axis_name="s")
    return pl.kernel(
        _agg_body,
        out_type=jax.ShapeDtypeStruct((NC, OWN_PAD, D), jnp.float32),
        mesh=mesh,
        scratch_types=[
            pltpu.VMEM((2, IBLK, CHUNK), jnp.int32),
            pltpu.VMEM((2, IBLK, CHUNK), jnp.int32),
            pltpu.VMEM((NBUF, CHUNK, D), jnp.float32),
            pltpu.VMEM_SHARED((ACC_ROWS, D), jnp.float32),
        ] + [pltpu.SemaphoreType.DMA] * (1 + 2 * NBUF),
    )(y, src3, dst3)


def _mm_body(x_ref, w_ref, degp_ref, y_ref):
    xw = jnp.dot(x_ref[...], w_ref[...], preferred_element_type=jnp.float32)
    deg = degp_ref[0, 0, :] + degp_ref[0, 1, :] + 1.0
    dis = lax.rsqrt(deg)
    y_ref[...] = xw * dis[:, None]


@jax.jit
def _mm_kernel(x, W, degp5):
    rb = 2000
    grid = N_NODES // rb
    return pl.pallas_call(
        _mm_body,
        grid=(grid,),
        in_specs=[
            pl.BlockSpec((rb, D), lambda j: (j, 0)),
            pl.BlockSpec((D, D), lambda j: (0, 0)),
            pl.BlockSpec((1, NC, rb), lambda j: (j, 0, 0)),
        ],
        out_specs=pl.BlockSpec((rb, D), lambda j: (j, 0)),
        out_shape=jax.ShapeDtypeStruct((N_NODES, D), jnp.float32),
    )(x, W, degp5)


def _fin_body(agg_ref, y_ref, degp_ref, b_ref, out_ref):
    deg = degp_ref[0, 0, :] + degp_ref[0, 1, :] + 1.0
    dis = lax.rsqrt(deg)
    out_ref[...] = jnp.maximum(
        dis[:, None] * (agg_ref[0] + y_ref[...]) + b_ref[...], 0.0)


@jax.jit
def _fin_kernel(agg, y, degp10, b2):
    rb = 1000
    half = OWN // rb
    return pl.pallas_call(
        _fin_body,
        grid=(N_NODES // rb,),
        in_specs=[
            pl.BlockSpec((1, rb, D), lambda j: (j // half, j % half, 0)),
            pl.BlockSpec((rb, D), lambda j: (j, 0)),
            pl.BlockSpec((1, NC, rb), lambda j: (j, 0, 0)),
            pl.BlockSpec((1, D), lambda j: (0, 0)),
        ],
        out_specs=pl.BlockSpec((rb, D), lambda j: (j, 0)),
        out_shape=jax.ShapeDtypeStruct((N_NODES, D), jnp.float32),
    )(agg, y, degp10, b2)


def kernel(x, edge_index, W, b):
    ei = edge_index.astype(jnp.int32)
    src4 = ei[0].reshape(NC, NS, NCHUNK, CHUNK)
    dst4 = ei[1].reshape(NC, NS, NCHUNK, CHUNK)
    src3 = ei[0].reshape(NS, NIB, IBLK, CHUNK)
    dst3 = ei[1].reshape(NS, NIB, IBLK, CHUNK)
    deg_part = _deg_kernel(dst4)
    degp = deg_part[:, :N_NODES]
    degp5 = jnp.swapaxes(degp.reshape(NC, 5, 2000), 0, 1)
    degp10 = jnp.swapaxes(degp.reshape(NC, 10, 1000), 0, 1)
    y = _mm_kernel(x, W, degp5)
    agg = _agg_kernel(y, src3, dst3)
    return _fin_kernel(agg, y, degp10, b.reshape(1, D))


# speedup vs baseline: 30.1692x; 1.2370x over previous
import functools

import jax
import jax.numpy as jnp
from jax import lax
from jax.experimental import pallas as pl
from jax.experimental.pallas import tpu as pltpu
from jax.experimental.pallas import tpu_sc as plsc

N_NODES = 10000
N_EDGES = 320000
D = 128

NC = 2
NS = 16
NW = NC * NS
E_PER_TILE = N_EDGES // NW
CHUNK = 80
NCHUNK = E_PER_TILE // CHUNK
N_PAD = 10240
STRIPE = N_PAD // NS
NCHUNK_F = N_EDGES // NS // CHUNK
IBLK = 25
NIB = NCHUNK_F // IBLK
NBUF = 5
PREF = 3
LAG = 1
OWN = 5000
OWN_PAD = 5120
ACC_ROWS = 5128
AGG_STRIPE = OWN_PAD // NS
ROW_CHUNK = 80


def _deg_body(dst_hbm, deg_hbm, idx_v, ones_v, stripe_v, acc_sh):
    c = lax.axis_index("c")
    s = lax.axis_index("s")
    for i in range(CHUNK // 16):
        ones_v[pl.ds(i * 16, 16)] = jnp.ones((16,), jnp.float32)
    for i in range(640 // 16):
        stripe_v[pl.ds(i * 16, 16)] = jnp.zeros((16,), jnp.float32)
    pltpu.sync_copy(stripe_v, acc_sh.at[pl.ds(s * 640, 640)])
    plsc.subcore_barrier()
    pltpu.sync_copy(dst_hbm.at[c, s], idx_v)

    def chunk(j, carry):
        pltpu.sync_copy(ones_v, acc_sh.at[idx_v.at[j]], add=True)
        return carry

    lax.fori_loop(0, NCHUNK, chunk, 0)
    plsc.subcore_barrier()
    pltpu.sync_copy(acc_sh.at[pl.ds(s * 640, 640)], stripe_v)
    pltpu.sync_copy(stripe_v, deg_hbm.at[c, pl.ds(s * 640, 640)])


@jax.jit
def _deg_kernel(dst4):
    mesh = plsc.VectorSubcoreMesh(core_axis_name="c", subcore_axis_name="s")
    return pl.kernel(
        _deg_body,
        out_type=jax.ShapeDtypeStruct((NC, N_PAD), jnp.float32),
        mesh=mesh,
        scratch_types=[
            pltpu.VMEM((NCHUNK, CHUNK), jnp.int32),
            pltpu.VMEM((CHUNK,), jnp.float32),
            pltpu.VMEM((640,), jnp.float32),
            pltpu.VMEM_SHARED((N_PAD,), jnp.float32),
        ],
    )(dst4)


def _agg_body(y_hbm, src_hbm, dst_hbm, agg_hbm,
              sidx_v, didx_v, rows_v, acc_sh, *sems):
    isem = sems[0]
    gsem = sems[1:1 + NBUF]
    ssem = sems[1 + NBUF:1 + 2 * NBUF]
    c = lax.axis_index("c")
    s = lax.axis_index("s")
    base = s * AGG_STRIPE
    lo = c * OWN

    def zrow(r, carry):
        for k in range(D // 16):
            rows_v[0, r, pl.ds(k * 16, 16)] = jnp.zeros((16,), jnp.float32)
        return carry

    lax.fori_loop(0, ROW_CHUNK, zrow, 0)
    for i in range(AGG_STRIPE // ROW_CHUNK):
        pltpu.sync_copy(rows_v.at[0],
                        acc_sh.at[pl.ds(base + i * ROW_CHUNK, ROW_CHUNK)])
    plsc.subcore_barrier()

    pltpu.async_copy(src_hbm.at[s, 0], sidx_v.at[0], isem)
    pltpu.async_copy(dst_hbm.at[s, 0], didx_v.at[0], isem)

    def block(ib, carry):
        ib2 = lax.rem(ib, 2)
        pltpu.make_async_copy(src_hbm.at[s, ib], sidx_v.at[ib2], isem).wait()
        pltpu.make_async_copy(dst_hbm.at[s, ib], didx_v.at[ib2], isem).wait()

        gh = {}
        for j in range(PREF):
            gh[j] = pltpu.async_copy(
                y_hbm.at[sidx_v.at[ib2, j]], rows_v.at[j % NBUF],
                gsem[j % NBUF])

        @pl.when(ib < NIB - 1)
        def _():
            nxt = lax.rem(ib + 1, 2)
            pltpu.async_copy(src_hbm.at[s, ib + 1], sidx_v.at[nxt], isem)
            pltpu.async_copy(dst_hbm.at[s, ib + 1], didx_v.at[nxt], isem)

        def remap(t, carry2):
            j = t // (CHUNK // 16)
            k = t % (CHUNK // 16)
            d = didx_v[ib2, j, pl.ds(k * 16, 16)] - lo
            ok = jnp.logical_and(d >= 0, d < OWN)
            dump = OWN_PAD + (lax.iota(jnp.int32, 16) & 7)
            didx_v[ib2, j, pl.ds(k * 16, 16)] = jnp.where(ok, d, dump)
            return carry2

        lax.fori_loop(0, IBLK * (CHUNK // 16), remap, 0)

        sh = {}
        for j in range(IBLK):
            b = j % NBUF
            gh[j].wait()
            sh[j] = pltpu.async_copy(
                rows_v.at[b], acc_sh.at[didx_v.at[ib2, j]], ssem[b], add=True)
            if j - LAG >= 0:
                sh[j - LAG].wait()
            nj = j + PREF
            if nj < IBLK:
                nb = nj % NBUF
                gh[nj] = pltpu.async_copy(
                    y_hbm.at[sidx_v.at[ib2, nj]], rows_v.at[nb], gsem[nb])
        for j in range(IBLK - LAG, IBLK):
            sh[j].wait()
        return carry

    lax.fori_loop(0, NIB, block, 0)
    plsc.subcore_barrier()
    for i in range(AGG_STRIPE // ROW_CHUNK):
        off = base + i * ROW_CHUNK
        pltpu.sync_copy(acc_sh.at[pl.ds(off, ROW_CHUNK)], rows_v.at[0])
        pltpu.sync_copy(rows_v.at[0], agg_hbm.at[c, pl.ds(off, ROW_CHUNK)])


@jax.jit
def _agg_kernel(y, src3, dst3):
    mesh = plsc.VectorSubcoreMesh(core_axis_name="c", subcore_axis_name="s")
    return pl.kernel(
        _agg_body,
        out_type=jax.ShapeDtypeStruct((NC, OWN_PAD, D), jnp.float32),
        mesh=mesh,
        scratch_types=[
            pltpu.VMEM((2, IBLK, CHUNK), jnp.int32),
            pltpu.VMEM((2, IBLK, CHUNK), jnp.int32),
            pltpu.VMEM((NBUF, CHUNK, D), jnp.float32),
            pltpu.VMEM_SHARED((ACC_ROWS, D), jnp.float32),
        ] + [pltpu.SemaphoreType.DMA] * (1 + 2 * NBUF),
    )(y, src3, dst3)


def _mm_body(x_ref, w_ref, degp_ref, y_ref):
    xw = jnp.dot(x_ref[...], w_ref[...], preferred_element_type=jnp.float32)
    deg = degp_ref[0, 0, :] + degp_ref[0, 1, :] + 1.0
    dis = lax.rsqrt(deg)
    y_ref[...] = xw * dis[:, None]


@jax.jit
def _mm_kernel(x, W, degp5):
    rb = 2000
    grid = N_NODES // rb
    return pl.pallas_call(
        _mm_body,
        grid=(grid,),
        in_specs=[
            pl.BlockSpec((rb, D), lambda j: (j, 0)),
            pl.BlockSpec((D, D), lambda j: (0, 0)),
            pl.BlockSpec((1, NC, rb), lambda j: (j, 0, 0)),
        ],
        out_specs=pl.BlockSpec((rb, D), lambda j: (j, 0)),
        out_shape=jax.ShapeDtypeStruct((N_NODES, D), jnp.float32),
    )(x, W, degp5)


def _fin_body(agg_ref, y_ref, degp_ref, b_ref, out_ref):
    deg = degp_ref[0, 0, :] + degp_ref[0, 1, :] + 1.0
    dis = lax.rsqrt(deg)
    out_ref[...] = jnp.maximum(
        dis[:, None] * (agg_ref[0] + y_ref[...]) + b_ref[...], 0.0)


@jax.jit
def _fin_kernel(agg, y, degp10, b2):
    rb = 1000
    half = OWN // rb
    return pl.pallas_call(
        _fin_body,
        grid=(N_NODES // rb,),
        in_specs=[
            pl.BlockSpec((1, rb, D), lambda j: (j // half, j % half, 0)),
            pl.BlockSpec((rb, D), lambda j: (j, 0)),
            pl.BlockSpec((1, NC, rb), lambda j: (j, 0, 0)),
            pl.BlockSpec((1, D), lambda j: (0, 0)),
        ],
        out_specs=pl.BlockSpec((rb, D), lambda j: (j, 0)),
        out_shape=jax.ShapeDtypeStruct((N_NODES, D), jnp.float32),
    )(agg, y, degp10, b2)


def kernel(x, edge_index, W, b):
    ei = edge_index.astype(jnp.int32)
    src4 = ei[0].reshape(NC, NS, NCHUNK, CHUNK)
    dst4 = ei[1].reshape(NC, NS, NCHUNK, CHUNK)
    src3 = ei[0].reshape(NS, NIB, IBLK, CHUNK)
    dst3 = ei[1].reshape(NS, NIB, IBLK, CHUNK)
    deg_part = _deg_kernel(dst4)
    degp = deg_part[:, :N_NODES]
    degp5 = jnp.swapaxes(degp.reshape(NC, 5, 2000), 0, 1)
    degp10 = jnp.swapaxes(degp.reshape(NC, 10, 1000), 0, 1)
    y = _mm_kernel(x, W, degp5)
    agg = _agg_kernel(y, src3, dst3)
    return _fin_kernel(agg, y, degp10, b.reshape(1, D))
